# baseline (speedup 1.0000x reference)
import jax
import jax.numpy as jnp
from jax import lax
from jax.experimental import pallas as pl
from jax.experimental.pallas import tpu as pltpu
from jax.experimental.pallas import tpu_sc as plsc

_INFO = plsc.get_sparse_core_info()
_NC = _INFO.num_cores
_NS = _INFO.num_subcores
_NW = _NC * _NS

_B = 4096 * 8
_D = 256
_BPW = _B // _NW
_C = 64
_NCHUNK = _BPW // _C
_NBUF = 4
_NSLOT = 3
_LAG = 3


def _gather_body(x_hbm, idx_hbm, out_hbm, idx_v, shared, *rest):
  bufs = rest[:_NBUF]
  gsems = rest[_NBUF:2 * _NBUF]
  asems = rest[2 * _NBUF:2 * _NBUF + _NSLOT]
  bsems = rest[2 * _NBUF + _NSLOT:]

  sid = lax.axis_index("s")
  wid = sid * _NC + lax.axis_index("c")
  base = wid * _BPW

  head = _LAG * _C
  pltpu.sync_copy(idx_hbm.at[pl.ds(base, head)], idx_v.at[pl.ds(0, head)])

  def start_gather(g):
    b = g % _NBUF
    return pltpu.async_copy(
        x_hbm.at[idx_v.at[pl.ds(g * _C, _C)]], bufs[b], gsems[b])

  def start_a(g):
    return pltpu.async_copy(
        bufs[g % _NBUF], shared.at[sid, g % _NSLOT], asems[g % _NSLOT])

  def start_b(g):
    return pltpu.async_copy(
        shared.at[sid, g % _NSLOT], out_hbm.at[pl.ds(base + g * _C, _C)],
        bsems[g % _NSLOT])

  gathers = [None] * _NCHUNK
  aw = [None] * _NCHUNK
  bw = [None] * _NCHUNK
  for g in range(min(_LAG, _NCHUNK)):
    gathers[g] = start_gather(g)
  pltpu.sync_copy(idx_hbm.at[pl.ds(base + head, _BPW - head)],
                  idx_v.at[pl.ds(head, _BPW - head)])
  for g in range(_NCHUNK):
    gathers[g].wait()
    if g >= 1:
      aw[g - 1].wait()
      bw[g - 1] = start_b(g - 1)
    if g - _NSLOT >= 0:
      bw[g - _NSLOT].wait()
    aw[g] = start_a(g)
    h = g + _LAG
    if h < _NCHUNK:
      gathers[h] = start_gather(h)
  aw[_NCHUNK - 1].wait()
  bw[_NCHUNK - 1] = start_b(_NCHUNK - 1)
  for g in range(max(0, _NCHUNK - _NSLOT), _NCHUNK):
    bw[g].wait()


@jax.jit
def _pooled_gather(x, idx):
  mesh = plsc.VectorSubcoreMesh(core_axis_name="c", subcore_axis_name="s")
  return pl.kernel(
      _gather_body,
      out_type=jax.ShapeDtypeStruct((_B, _D), jnp.float32),
      mesh=mesh,
      scratch_types=(
          [pltpu.VMEM((_BPW,), jnp.int32)]
          + [pltpu.VMEM_SHARED((_NS, _NSLOT, _C, _D), jnp.float32)]
          + [pltpu.VMEM((_C, _D), jnp.float32)] * _NBUF
          + [pltpu.SemaphoreType.DMA] * (_NBUF + 2 * _NSLOT)
      ),
  )(x, idx)


def kernel(x, pooling_mask):
  return _pooled_gather(x, pooling_mask.reshape(-1))


# speedup vs baseline: 1.0045x; 1.0045x over previous
import jax
import jax.numpy as jnp
from jax import lax
from jax.experimental import pallas as pl
from jax.experimental.pallas import tpu as pltpu
from jax.experimental.pallas import tpu_sc as plsc

_INFO = plsc.get_sparse_core_info()
_NC = _INFO.num_cores
_NS = _INFO.num_subcores
_NW = _NC * _NS

_B = 4096 * 8
_D = 256
_BPW = _B // _NW
_C = 64
_NCHUNK = _BPW // _C
_NBUF = 6
_LAG = 4


def _gather_body(x_hbm, idx_hbm, out_hbm, idx_v, *rest):
  bufs = rest[:_NBUF]
  gsems = rest[_NBUF:2 * _NBUF]
  osems = rest[2 * _NBUF:]

  wid = lax.axis_index("s") * _NC + lax.axis_index("c")
  base = wid * _BPW

  head = _LAG * _C
  pltpu.sync_copy(idx_hbm.at[pl.ds(base, head)], idx_v.at[pl.ds(0, head)])

  def start_gather(g):
    b = g % _NBUF
    return pltpu.async_copy(
        x_hbm.at[idx_v.at[pl.ds(g * _C, _C)]], bufs[b], gsems[b])

  gathers = [None] * _NCHUNK
  writes = [None] * _NCHUNK
  for g in range(min(_LAG, _NCHUNK)):
    gathers[g] = start_gather(g)
  pltpu.sync_copy(idx_hbm.at[pl.ds(base + head, _BPW - head)],
                  idx_v.at[pl.ds(head, _BPW - head)])
  for g in range(_NCHUNK):
    gathers[g].wait()
    writes[g] = pltpu.async_copy(
        bufs[g % _NBUF], out_hbm.at[pl.ds(base + g * _C, _C)],
        osems[g % _NBUF])
    h = g + _LAG
    if h < _NCHUNK:
      if h - _NBUF >= 0:
        writes[h - _NBUF].wait()
      gathers[h] = start_gather(h)
  for g in range(max(0, _NCHUNK - _NBUF), _NCHUNK):
    writes[g].wait()


@jax.jit
def _pooled_gather(x, idx):
  mesh = plsc.VectorSubcoreMesh(core_axis_name="c", subcore_axis_name="s")
  return pl.kernel(
      _gather_body,
      out_type=jax.ShapeDtypeStruct((_B, _D), jnp.float32),
      mesh=mesh,
      scratch_types=(
          [pltpu.VMEM((_BPW,), jnp.int32)]
          + [pltpu.VMEM((_C, _D), jnp.float32)] * _NBUF
          + [pltpu.SemaphoreType.DMA] * (2 * _NBUF)
      ),
  )(x, idx)


def kernel(x, pooling_mask):
  return _pooled_gather(x, pooling_mask.reshape(-1))


# speedup vs baseline: 1.0194x; 1.0149x over previous
import jax
import jax.numpy as jnp
from jax import lax
from jax.experimental import pallas as pl
from jax.experimental.pallas import tpu as pltpu
from jax.experimental.pallas import tpu_sc as plsc

_INFO = plsc.get_sparse_core_info()
_NC = _INFO.num_cores
_NS = _INFO.num_subcores
_NW = _NC * _NS

_B = 4096 * 8
_D = 256
_BPW = _B // _NW
_C = 64
_NCHUNK = _BPW // _C
_NBUF = 6
_LAG = 5


def _gather_body(x_hbm, idx_hbm, out_hbm, idx_v, *rest):
  bufs = rest[:_NBUF]
  gsems = rest[_NBUF:2 * _NBUF]
  osems = rest[2 * _NBUF:]

  wid = lax.axis_index("s") * _NC + lax.axis_index("c")
  base = wid * _BPW

  head = _LAG * _C
  pltpu.sync_copy(idx_hbm.at[pl.ds(base, head)], idx_v.at[pl.ds(0, head)])

  def start_gather(g):
    b = g % _NBUF
    return pltpu.async_copy(
        x_hbm.at[idx_v.at[pl.ds(g * _C, _C)]], bufs[b], gsems[b])

  gathers = [None] * _NCHUNK
  writes = [None] * _NCHUNK
  for g in range(min(_LAG, _NCHUNK)):
    gathers[g] = start_gather(g)
  pltpu.sync_copy(idx_hbm.at[pl.ds(base + head, _BPW - head)],
                  idx_v.at[pl.ds(head, _BPW - head)])
  for g in range(_NCHUNK):
    gathers[g].wait()
    writes[g] = pltpu.async_copy(
        bufs[g % _NBUF], out_hbm.at[pl.ds(base + g * _C, _C)],
        osems[g % _NBUF])
    h = g + _LAG
    if h < _NCHUNK:
      if h - _NBUF >= 0:
        writes[h - _NBUF].wait()
      gathers[h] = start_gather(h)
  for g in range(max(0, _NCHUNK - _NBUF), _NCHUNK):
    writes[g].wait()


@jax.jit
def _pooled_gather(x, idx):
  mesh = plsc.VectorSubcoreMesh(core_axis_name="c", subcore_axis_name="s")
  return pl.kernel(
      _gather_body,
      out_type=jax.ShapeDtypeStruct((_B, _D), jnp.float32),
      mesh=mesh,
      scratch_types=(
          [pltpu.VMEM((_BPW,), jnp.int32)]
          + [pltpu.VMEM((_C, _D), jnp.float32)] * _NBUF
          + [pltpu.SemaphoreType.DMA] * (2 * _NBUF)
      ),
  )(x, idx)


def kernel(x, pooling_mask):
  return _pooled_gather(x, pooling_mask.reshape(-1))


# speedup vs baseline: 1.0227x; 1.0033x over previous
import jax
import jax.numpy as jnp
from jax import lax
from jax.experimental import pallas as pl
from jax.experimental.pallas import tpu as pltpu
from jax.experimental.pallas import tpu_sc as plsc

_INFO = plsc.get_sparse_core_info()
_NC = _INFO.num_cores
_NS = _INFO.num_subcores
_NW = _NC * _NS

_B = 4096 * 8
_D = 256
_BPW = _B // _NW
_C = 64
_NCHUNK = _BPW // _C
_NBUF = 7
_LAG = 6


def _gather_body(x_hbm, idx_hbm, out_hbm, idx_v, *rest):
  bufs = rest[:_NBUF]
  gsems = rest[_NBUF:2 * _NBUF]
  osems = rest[2 * _NBUF:]

  wid = lax.axis_index("s") * _NC + lax.axis_index("c")
  base = wid * _BPW

  head = _LAG * _C
  pltpu.sync_copy(idx_hbm.at[pl.ds(base, head)], idx_v.at[pl.ds(0, head)])

  def start_gather(g):
    b = g % _NBUF
    return pltpu.async_copy(
        x_hbm.at[idx_v.at[pl.ds(g * _C, _C)]], bufs[b], gsems[b])

  gathers = [None] * _NCHUNK
  writes = [None] * _NCHUNK
  for g in range(min(_LAG, _NCHUNK)):
    gathers[g] = start_gather(g)
  pltpu.sync_copy(idx_hbm.at[pl.ds(base + head, _BPW - head)],
                  idx_v.at[pl.ds(head, _BPW - head)])
  for g in range(_NCHUNK):
    gathers[g].wait()
    writes[g] = pltpu.async_copy(
        bufs[g % _NBUF], out_hbm.at[pl.ds(base + g * _C, _C)],
        osems[g % _NBUF])
    h = g + _LAG
    if h < _NCHUNK:
      if h - _NBUF >= 0:
        writes[h - _NBUF].wait()
      gathers[h] = start_gather(h)
  for g in range(max(0, _NCHUNK - _NBUF), _NCHUNK):
    writes[g].wait()


@jax.jit
def _pooled_gather(x, idx):
  mesh = plsc.VectorSubcoreMesh(core_axis_name="c", subcore_axis_name="s")
  return pl.kernel(
      _gather_body,
      out_type=jax.ShapeDtypeStruct((_B, _D), jnp.float32),
      mesh=mesh,
      scratch_types=(
          [pltpu.VMEM((_BPW,), jnp.int32)]
          + [pltpu.VMEM((_C, _D), jnp.float32)] * _NBUF
          + [pltpu.SemaphoreType.DMA] * (2 * _NBUF)
      ),
  )(x, idx)


def kernel(x, pooling_mask):
  return _pooled_gather(x, pooling_mask.reshape(-1))


# speedup vs baseline: 1.0228x; 1.0001x over previous
import jax
import jax.numpy as jnp
from jax import lax
from jax.experimental import pallas as pl
from jax.experimental.pallas import tpu as pltpu
from jax.experimental.pallas import tpu_sc as plsc

_INFO = plsc.get_sparse_core_info()
_NC = _INFO.num_cores
_NS = _INFO.num_subcores
_NW = _NC * _NS

_B = 4096 * 8
_D = 256
_BPW = _B // _NW
_C = 32
_NCHUNK = _BPW // _C
_NBUF = 14
_LAG = 12


def _gather_body(x_hbm, idx_hbm, out_hbm, idx_v, *rest):
  bufs = rest[:_NBUF]
  gsems = rest[_NBUF:2 * _NBUF]
  osems = rest[2 * _NBUF:]

  wid = lax.axis_index("s") * _NC + lax.axis_index("c")
  base = wid * _BPW

  head = _LAG * _C
  pltpu.sync_copy(idx_hbm.at[pl.ds(base, head)], idx_v.at[pl.ds(0, head)])

  def start_gather(g):
    b = g % _NBUF
    return pltpu.async_copy(
        x_hbm.at[idx_v.at[pl.ds(g * _C, _C)]], bufs[b], gsems[b])

  gathers = [None] * _NCHUNK
  writes = [None] * _NCHUNK
  for g in range(min(_LAG, _NCHUNK)):
    gathers[g] = start_gather(g)
  pltpu.sync_copy(idx_hbm.at[pl.ds(base + head, _BPW - head)],
                  idx_v.at[pl.ds(head, _BPW - head)])
  for g in range(_NCHUNK):
    gathers[g].wait()
    writes[g] = pltpu.async_copy(
        bufs[g % _NBUF], out_hbm.at[pl.ds(base + g * _C, _C)],
        osems[g % _NBUF])
    h = g + _LAG
    if h < _NCHUNK:
      if h - _NBUF >= 0:
        writes[h - _NBUF].wait()
      gathers[h] = start_gather(h)
  for g in range(max(0, _NCHUNK - _NBUF), _NCHUNK):
    writes[g].wait()


@jax.jit
def _pooled_gather(x, idx):
  mesh = plsc.VectorSubcoreMesh(core_axis_name="c", subcore_axis_name="s")
  return pl.kernel(
      _gather_body,
      out_type=jax.ShapeDtypeStruct((_B, _D), jnp.float32),
      mesh=mesh,
      scratch_types=(
          [pltpu.VMEM((_BPW,), jnp.int32)]
          + [pltpu.VMEM((_C, _D), jnp.float32)] * _NBUF
          + [pltpu.SemaphoreType.DMA] * (2 * _NBUF)
      ),
  )(x, idx)


def kernel(x, pooling_mask):
  return _pooled_gather(x, pooling_mask.reshape(-1))
